# Initial kernel scaffold; baseline (speedup 1.0000x reference)
import jax
import jax.numpy as jnp
from jax.experimental import pallas as pl


def kernel(x, edge_index, batch, W1, b1, W2, b2, W3, b3, Wp1, bp1, Wp2, bp2):
    raise NotImplementedError("write your pallas kernel here")


# speedup vs baseline: 16.8879x; 16.8879x over previous
import functools

import jax
import jax.numpy as jnp
from jax import lax
from jax.experimental import pallas as pl
from jax.experimental.pallas import tpu as pltpu
from jax.experimental.pallas import tpu_sc as plsc

_NC = 2
_NS = 16
_NW = _NC * _NS
_CH = 80


def _mesh():
    return plsc.VectorSubcoreMesh(core_axis_name="c", subcore_axis_name="s")


def _make_deg(N, E):
    ET = E // _NW
    NCH = ET // _CH
    ROWS = -(-N // (128 * _NS)) * _NS
    RT = ROWS // _NS

    @functools.partial(
        pl.kernel,
        mesh=_mesh(),
        out_type=jax.ShapeDtypeStruct((_NW, ROWS, 128), jnp.float32),
        compiler_params=pltpu.CompilerParams(needs_layout_passes=False),
        scratch_types=[
            pltpu.VMEM((ROWS * 128,), jnp.float32),
            pltpu.VMEM((ROWS, 128), jnp.float32),
            pltpu.VMEM((NCH, _CH), jnp.int32),
        ],
    )
    def deg_kernel(ei_hbm, out_hbm, hist, hist2, didx):
        c = lax.axis_index("c")
        s = lax.axis_index("s")
        wid = c * _NS + s

        zeros16 = jnp.zeros((16,), jnp.float32)
        lanes = lax.iota(jnp.int32, 16)

        def zero_body(r, carry):
            for k in range(8):
                hist[pl.ds(r * 128 + k * 16, 16)] = zeros16
            return carry

        lax.fori_loop(0, ROWS, zero_body, 0)

        pltpu.sync_copy(ei_hbm.at[1, wid], didx)

        ones16 = jnp.ones((16,), jnp.float32)

        def chunk_body(i, carry):
            for k in range(_CH // 16):
                idx = didx[i, pl.ds(k * 16, 16)]
                for j in range(16):
                    plsc.addupdate_scatter(
                        hist, [idx], ones16, mask=lanes == j)
            return carry

        lax.fori_loop(0, NCH, chunk_body, 0)

        def pack_body(r, carry):
            for k in range(8):
                hist2[r, pl.ds(k * 16, 16)] = hist[pl.ds(r * 128 + k * 16, 16)]
            return carry

        lax.fori_loop(0, ROWS, pack_body, 0)
        pltpu.sync_copy(hist2, out_hbm.at[wid])

    return deg_kernel, ROWS * 128


def _make_agg(N, D, E):
    ET = E // _NW
    NCH = ET // _CH
    RS0 = -(-N // (8 * _NS)) * 8
    RSL = N - (_NS - 1) * RS0

    @functools.partial(
        pl.kernel,
        mesh=_mesh(),
        out_type=jax.ShapeDtypeStruct((_NC, N, D), jnp.float32),
        scratch_types=[
            pltpu.VMEM_SHARED((N, D), jnp.float32),
            pltpu.VMEM((2, NCH, _CH), jnp.int32),
            pltpu.VMEM((_CH, D), jnp.float32),
            pltpu.SemaphoreType.DMA,
        ],
    )
    def agg_kernel(g_hbm, ei_hbm, out_hbm, acc, eidx, rows, sem):
        c = lax.axis_index("c")
        s = lax.axis_index("s")
        wid = c * _NS + s

        @pl.when(s < _NS - 1)
        def _():
            r0 = pl.multiple_of(s * RS0, 8)
            pltpu.sync_copy(g_hbm.at[pl.ds(r0, RS0)], acc.at[pl.ds(r0, RS0)])

        @pl.when(s == _NS - 1)
        def _():
            r0 = (_NS - 1) * RS0
            pltpu.sync_copy(g_hbm.at[pl.ds(r0, RSL)], acc.at[pl.ds(r0, RSL)])

        pltpu.sync_copy(ei_hbm.at[:, wid], eidx)
        plsc.subcore_barrier()

        def chunk_body(i, carry):
            pltpu.async_copy(g_hbm.at[eidx.at[0, i]], rows, sem).wait()
            pltpu.sync_copy(rows, acc.at[eidx.at[1, i]], add=True)
            return carry

        lax.fori_loop(0, NCH, chunk_body, 0)

        plsc.subcore_barrier()

        @pl.when(s < _NS - 1)
        def _():
            r0 = pl.multiple_of(s * RS0, 8)
            pltpu.sync_copy(acc.at[pl.ds(r0, RS0)],
                            out_hbm.at[c, pl.ds(r0, RS0)])

        @pl.when(s == _NS - 1)
        def _():
            r0 = (_NS - 1) * RS0
            pltpu.sync_copy(acc.at[pl.ds(r0, RSL)],
                            out_hbm.at[c, pl.ds(r0, RSL)])

    return agg_kernel


_PREC = jax.lax.Precision.HIGHEST


def _tc_pre(x, W, cnt, R=2000):
    N, D = x.shape
    H = W.shape[1]

    def body(x_ref, w_ref, c_ref, g_ref):
        dis = lax.rsqrt(c_ref[...] + 1.0)
        h = jnp.dot(x_ref[...], w_ref[...],
                    preferred_element_type=jnp.float32, precision=_PREC)
        g_ref[...] = h * dis

    return pl.pallas_call(
        body,
        grid=(N // R,),
        in_specs=[
            pl.BlockSpec((R, D), lambda i: (i, 0)),
            pl.BlockSpec((D, H), lambda i: (0, 0)),
            pl.BlockSpec((R, 1), lambda i: (i, 0)),
        ],
        out_specs=pl.BlockSpec((R, H), lambda i: (i, 0)),
        out_shape=jax.ShapeDtypeStruct((N, H), jnp.float32),
    )(x, W, cnt)


def _tc_mid(a, g, cnt, b, W, R=2000):
    _, N, H = a.shape
    H2 = W.shape[1]

    def body(a_ref, g_ref, c_ref, b_ref, w_ref, o_ref):
        dis = lax.rsqrt(c_ref[...] + 1.0)
        z = (a_ref[0] + a_ref[1] - g_ref[...]) * dis + b_ref[...]
        h = jnp.maximum(z, 0.0)
        o_ref[...] = jnp.dot(h, w_ref[...],
                             preferred_element_type=jnp.float32,
                             precision=_PREC) * dis

    return pl.pallas_call(
        body,
        grid=(N // R,),
        in_specs=[
            pl.BlockSpec((2, R, H), lambda i: (0, i, 0)),
            pl.BlockSpec((R, H), lambda i: (i, 0)),
            pl.BlockSpec((R, 1), lambda i: (i, 0)),
            pl.BlockSpec((1, H), lambda i: (0, 0)),
            pl.BlockSpec((H, H2), lambda i: (0, 0)),
        ],
        out_specs=pl.BlockSpec((R, H2), lambda i: (i, 0)),
        out_shape=jax.ShapeDtypeStruct((N, H2), jnp.float32),
    )(a, g, cnt, b, W)


def _tc_post(a, g, cnt, b, Wp1, bp1, Wp2, bp2, R=2000):
    _, N, H = a.shape
    O = Wp2.shape[1]

    def body(a_ref, g_ref, c_ref, b_ref, w1_ref, b1_ref, w2_ref, b2_ref,
             emb_ref, lsm_ref):
        dis = lax.rsqrt(c_ref[...] + 1.0)
        z = (a_ref[0] + a_ref[1] - g_ref[...]) * dis + b_ref[...]
        emb_ref[...] = z
        h = jnp.maximum(z, 0.0)
        t = jnp.dot(h, w1_ref[...], preferred_element_type=jnp.float32,
                    precision=_PREC) + b1_ref[...]
        o = jnp.dot(t, w2_ref[...], preferred_element_type=jnp.float32,
                    precision=_PREC) + b2_ref[...]
        m = jnp.max(o, axis=1, keepdims=True)
        lse = m + jnp.log(jnp.sum(jnp.exp(o - m), axis=1, keepdims=True))
        lsm_ref[...] = o - lse

    return pl.pallas_call(
        body,
        grid=(N // R,),
        in_specs=[
            pl.BlockSpec((2, R, H), lambda i: (0, i, 0)),
            pl.BlockSpec((R, H), lambda i: (i, 0)),
            pl.BlockSpec((R, 1), lambda i: (i, 0)),
            pl.BlockSpec((1, H), lambda i: (0, 0)),
            pl.BlockSpec((H, H), lambda i: (0, 0)),
            pl.BlockSpec((1, H), lambda i: (0, 0)),
            pl.BlockSpec((H, O), lambda i: (0, 0)),
            pl.BlockSpec((1, O), lambda i: (0, 0)),
        ],
        out_specs=[
            pl.BlockSpec((R, H), lambda i: (i, 0)),
            pl.BlockSpec((R, O), lambda i: (i, 0)),
        ],
        out_shape=[
            jax.ShapeDtypeStruct((N, H), jnp.float32),
            jax.ShapeDtypeStruct((N, O), jnp.float32),
        ],
    )(a, g, cnt, b, Wp1, bp1, Wp2, bp2)


def kernel(x, edge_index, batch, W1, b1, W2, b2, W3, b3, Wp1, bp1, Wp2, bp2):
    N, D = x.shape
    E = edge_index.shape[1]
    ET = E // _NW
    NCH = ET // _CH

    ei4 = edge_index.reshape(2, _NW, NCH, _CH)

    deg_kernel, P = _make_deg(N, E)
    agg_kernel = _make_agg(N, D, E)

    cnt2 = deg_kernel(ei4)
    cnt = cnt2.sum(axis=0).reshape(-1)[:N].reshape(N, 1)

    g1 = _tc_pre(x, W1, cnt)
    a1 = agg_kernel(g1, ei4)
    g2 = _tc_mid(a1, g1, cnt, b1.reshape(1, -1), W2)
    a2 = agg_kernel(g2, ei4)
    g3 = _tc_mid(a2, g2, cnt, b2.reshape(1, -1), W3)
    a3 = agg_kernel(g3, ei4)
    emb, lsm = _tc_post(a3, g3, cnt, b3.reshape(1, -1),
                        Wp1, bp1.reshape(1, -1), Wp2, bp2.reshape(1, -1))
    return emb, lsm


# speedup vs baseline: 22.5684x; 1.3364x over previous
import functools

import jax
import jax.numpy as jnp
from jax import lax
from jax.experimental import pallas as pl
from jax.experimental.pallas import tpu as pltpu
from jax.experimental.pallas import tpu_sc as plsc

_NC = 2
_NS = 16
_NW = _NC * _NS
_CH = 80
_CHA = 40
_SB = 50
_NB = 5


def _mesh():
    return plsc.VectorSubcoreMesh(core_axis_name="c", subcore_axis_name="s")


def _make_deg(N, E):
    ET = E // _NW
    NCH = ET // _CH
    ROWS = -(-N // (128 * _NS)) * _NS
    RT = ROWS // _NS

    @functools.partial(
        pl.kernel,
        mesh=_mesh(),
        out_type=jax.ShapeDtypeStruct((_NW, ROWS, 128), jnp.float32),
        compiler_params=pltpu.CompilerParams(needs_layout_passes=False),
        scratch_types=[
            pltpu.VMEM((ROWS * 128,), jnp.float32),
            pltpu.VMEM((ROWS, 128), jnp.float32),
            pltpu.VMEM((NCH, _CH), jnp.int32),
        ],
    )
    def deg_kernel(ei_hbm, out_hbm, hist, hist2, didx):
        c = lax.axis_index("c")
        s = lax.axis_index("s")
        wid = c * _NS + s

        zeros16 = jnp.zeros((16,), jnp.float32)
        lanes = lax.iota(jnp.int32, 16)

        def zero_body(r, carry):
            for k in range(8):
                hist[pl.ds(r * 128 + k * 16, 16)] = zeros16
            return carry

        lax.fori_loop(0, ROWS, zero_body, 0)

        pltpu.sync_copy(ei_hbm.at[1, wid], didx)

        ones16 = jnp.ones((16,), jnp.float32)

        def chunk_body(i, carry):
            for k in range(_CH // 16):
                idx = didx[i, pl.ds(k * 16, 16)]
                for j in range(16):
                    plsc.addupdate_scatter(
                        hist, [idx], ones16, mask=lanes == j)
            return carry

        lax.fori_loop(0, NCH, chunk_body, 0)

        def pack_body(r, carry):
            for k in range(8):
                hist2[r, pl.ds(k * 16, 16)] = hist[pl.ds(r * 128 + k * 16, 16)]
            return carry

        lax.fori_loop(0, ROWS, pack_body, 0)
        pltpu.sync_copy(hist2, out_hbm.at[wid])

    return deg_kernel, ROWS * 128


def _make_agg(N, D, E):
    ET = E // _NW
    NCH = ET // _CHA
    NSB = NCH // _SB
    RS0 = -(-N // (8 * _NS)) * 8
    RSL = N - (_NS - 1) * RS0

    @functools.partial(
        pl.kernel,
        mesh=_mesh(),
        out_type=jax.ShapeDtypeStruct((_NC, N, D), jnp.float32),
        scratch_types=[
            pltpu.VMEM_SHARED((N, D), jnp.float32),
            pltpu.VMEM((2, _SB, _CHA), jnp.int32),
            pltpu.VMEM((_NB, _CHA, D), jnp.float32),
        ] + [pltpu.SemaphoreType.DMA] * (2 * _NB),
    )
    def agg_kernel(g_hbm, ei_hbm, out_hbm, acc, eidx, rows, *sems):
        gsem = sems[:_NB]
        ssem = sems[_NB:]
        c = lax.axis_index("c")
        s = lax.axis_index("s")
        wid = c * _NS + s

        @pl.when(s < _NS - 1)
        def _():
            r0 = pl.multiple_of(s * RS0, 8)
            pltpu.sync_copy(g_hbm.at[pl.ds(r0, RS0)], acc.at[pl.ds(r0, RS0)])

        @pl.when(s == _NS - 1)
        def _():
            r0 = (_NS - 1) * RS0
            pltpu.sync_copy(g_hbm.at[pl.ds(r0, RSL)], acc.at[pl.ds(r0, RSL)])

        plsc.subcore_barrier()

        for sb in range(NSB):
            pltpu.sync_copy(ei_hbm.at[:, wid, sb], eidx)
            pltpu.async_copy(g_hbm.at[eidx.at[0, 0]], rows.at[0], gsem[0])

            def chunk_body(j, carry):
                for b in range(_NB):
                    i = j * _NB + b
                    bn = (b + 1) % _NB

                    @pl.when(i + 1 < _SB)
                    def _():
                        @pl.when(i + 1 >= _NB)
                        def _():
                            pltpu.make_async_copy(
                                rows.at[bn], acc.at[eidx.at[1, i + 1 - _NB]],
                                ssem[bn]).wait()
                        pltpu.async_copy(
                            g_hbm.at[eidx.at[0, i + 1]], rows.at[bn],
                            gsem[bn])

                    pltpu.make_async_copy(
                        g_hbm.at[eidx.at[0, i]], rows.at[b], gsem[b]).wait()
                    pltpu.async_copy(
                        rows.at[b], acc.at[eidx.at[1, i]], ssem[b], add=True)
                return carry

            lax.fori_loop(0, _SB // _NB, chunk_body, 0)

            for b in range(_NB):
                pltpu.make_async_copy(
                    rows.at[b], acc.at[eidx.at[1, _SB - _NB + b]],
                    ssem[b]).wait()

        plsc.subcore_barrier()

        @pl.when(s < _NS - 1)
        def _():
            r0 = pl.multiple_of(s * RS0, 8)
            pltpu.sync_copy(acc.at[pl.ds(r0, RS0)],
                            out_hbm.at[c, pl.ds(r0, RS0)])

        @pl.when(s == _NS - 1)
        def _():
            r0 = (_NS - 1) * RS0
            pltpu.sync_copy(acc.at[pl.ds(r0, RSL)],
                            out_hbm.at[c, pl.ds(r0, RSL)])

    return agg_kernel


_PREC = jax.lax.Precision.HIGHEST


def _tc_pre(x, W, cnt, R=2000):
    N, D = x.shape
    H = W.shape[1]

    def body(x_ref, w_ref, c_ref, g_ref):
        dis = lax.rsqrt(c_ref[...] + 1.0)
        h = jnp.dot(x_ref[...], w_ref[...],
                    preferred_element_type=jnp.float32, precision=_PREC)
        g_ref[...] = h * dis

    return pl.pallas_call(
        body,
        grid=(N // R,),
        in_specs=[
            pl.BlockSpec((R, D), lambda i: (i, 0)),
            pl.BlockSpec((D, H), lambda i: (0, 0)),
            pl.BlockSpec((R, 1), lambda i: (i, 0)),
        ],
        out_specs=pl.BlockSpec((R, H), lambda i: (i, 0)),
        out_shape=jax.ShapeDtypeStruct((N, H), jnp.float32),
    )(x, W, cnt)


def _tc_mid(a, g, cnt, b, W, R=2000):
    _, N, H = a.shape
    H2 = W.shape[1]

    def body(a_ref, g_ref, c_ref, b_ref, w_ref, o_ref):
        dis = lax.rsqrt(c_ref[...] + 1.0)
        z = (a_ref[0] + a_ref[1] - g_ref[...]) * dis + b_ref[...]
        h = jnp.maximum(z, 0.0)
        o_ref[...] = jnp.dot(h, w_ref[...],
                             preferred_element_type=jnp.float32,
                             precision=_PREC) * dis

    return pl.pallas_call(
        body,
        grid=(N // R,),
        in_specs=[
            pl.BlockSpec((2, R, H), lambda i: (0, i, 0)),
            pl.BlockSpec((R, H), lambda i: (i, 0)),
            pl.BlockSpec((R, 1), lambda i: (i, 0)),
            pl.BlockSpec((1, H), lambda i: (0, 0)),
            pl.BlockSpec((H, H2), lambda i: (0, 0)),
        ],
        out_specs=pl.BlockSpec((R, H2), lambda i: (i, 0)),
        out_shape=jax.ShapeDtypeStruct((N, H2), jnp.float32),
    )(a, g, cnt, b, W)


def _tc_post(a, g, cnt, b, Wp1, bp1, Wp2, bp2, R=2000):
    _, N, H = a.shape
    O = Wp2.shape[1]

    def body(a_ref, g_ref, c_ref, b_ref, w1_ref, b1_ref, w2_ref, b2_ref,
             emb_ref, lsm_ref):
        dis = lax.rsqrt(c_ref[...] + 1.0)
        z = (a_ref[0] + a_ref[1] - g_ref[...]) * dis + b_ref[...]
        emb_ref[...] = z
        h = jnp.maximum(z, 0.0)
        t = jnp.dot(h, w1_ref[...], preferred_element_type=jnp.float32,
                    precision=_PREC) + b1_ref[...]
        o = jnp.dot(t, w2_ref[...], preferred_element_type=jnp.float32,
                    precision=_PREC) + b2_ref[...]
        m = jnp.max(o, axis=1, keepdims=True)
        lse = m + jnp.log(jnp.sum(jnp.exp(o - m), axis=1, keepdims=True))
        lsm_ref[...] = o - lse

    return pl.pallas_call(
        body,
        grid=(N // R,),
        in_specs=[
            pl.BlockSpec((2, R, H), lambda i: (0, i, 0)),
            pl.BlockSpec((R, H), lambda i: (i, 0)),
            pl.BlockSpec((R, 1), lambda i: (i, 0)),
            pl.BlockSpec((1, H), lambda i: (0, 0)),
            pl.BlockSpec((H, H), lambda i: (0, 0)),
            pl.BlockSpec((1, H), lambda i: (0, 0)),
            pl.BlockSpec((H, O), lambda i: (0, 0)),
            pl.BlockSpec((1, O), lambda i: (0, 0)),
        ],
        out_specs=[
            pl.BlockSpec((R, H), lambda i: (i, 0)),
            pl.BlockSpec((R, O), lambda i: (i, 0)),
        ],
        out_shape=[
            jax.ShapeDtypeStruct((N, H), jnp.float32),
            jax.ShapeDtypeStruct((N, O), jnp.float32),
        ],
    )(a, g, cnt, b, Wp1, bp1, Wp2, bp2)


def kernel(x, edge_index, batch, W1, b1, W2, b2, W3, b3, Wp1, bp1, Wp2, bp2):
    N, D = x.shape
    E = edge_index.shape[1]
    ET = E // _NW

    ei4 = edge_index.reshape(2, _NW, ET // _CH, _CH)
    ei5 = edge_index.reshape(2, _NW, ET // (_SB * _CHA), _SB, _CHA)

    deg_kernel, P = _make_deg(N, E)
    agg_kernel = _make_agg(N, D, E)

    cnt2 = deg_kernel(ei4)
    cnt = cnt2.sum(axis=0).reshape(-1)[:N].reshape(N, 1)

    g1 = _tc_pre(x, W1, cnt)
    a1 = agg_kernel(g1, ei5)
    g2 = _tc_mid(a1, g1, cnt, b1.reshape(1, -1), W2)
    a2 = agg_kernel(g2, ei5)
    g3 = _tc_mid(a2, g2, cnt, b2.reshape(1, -1), W3)
    a3 = agg_kernel(g3, ei5)
    emb, lsm = _tc_post(a3, g3, cnt, b3.reshape(1, -1),
                        Wp1, bp1.reshape(1, -1), Wp2, bp2.reshape(1, -1))
    return emb, lsm


# speedup vs baseline: 26.4944x; 1.1740x over previous
import functools

import jax
import jax.numpy as jnp
from jax import lax
from jax.experimental import pallas as pl
from jax.experimental.pallas import tpu as pltpu
from jax.experimental.pallas import tpu_sc as plsc

_NC = 2
_NS = 16
_NW = _NC * _NS
_CH = 80
_CHA = 40
_SB = 25
_NB = 5


def _mesh():
    return plsc.VectorSubcoreMesh(core_axis_name="c", subcore_axis_name="s")


def _make_deg(N, E):
    ET = E // _NW
    NCH = ET // _CH
    ROWS = -(-N // (128 * _NS)) * _NS
    RT = ROWS // _NS

    @functools.partial(
        pl.kernel,
        mesh=_mesh(),
        out_type=jax.ShapeDtypeStruct((_NW, ROWS, 128), jnp.float32),
        compiler_params=pltpu.CompilerParams(needs_layout_passes=False),
        scratch_types=[
            pltpu.VMEM((ROWS * 128,), jnp.float32),
            pltpu.VMEM((ROWS, 128), jnp.float32),
            pltpu.VMEM((NCH, _CH), jnp.int32),
        ],
    )
    def deg_kernel(ei_hbm, out_hbm, hist, hist2, didx):
        c = lax.axis_index("c")
        s = lax.axis_index("s")
        wid = c * _NS + s

        zeros16 = jnp.zeros((16,), jnp.float32)
        lanes = lax.iota(jnp.int32, 16)

        def zero_body(r, carry):
            for k in range(8):
                hist[pl.ds(r * 128 + k * 16, 16)] = zeros16
            return carry

        lax.fori_loop(0, ROWS, zero_body, 0)

        pltpu.sync_copy(ei_hbm.at[1, wid], didx)

        ones16 = jnp.ones((16,), jnp.float32)

        def chunk_body(i, carry):
            for k in range(_CH // 16):
                idx = didx[i, pl.ds(k * 16, 16)]
                for j in range(16):
                    plsc.addupdate_scatter(
                        hist, [idx], ones16, mask=lanes == j)
            return carry

        lax.fori_loop(0, NCH, chunk_body, 0)

        def pack_body(r, carry):
            for k in range(8):
                hist2[r, pl.ds(k * 16, 16)] = hist[pl.ds(r * 128 + k * 16, 16)]
            return carry

        lax.fori_loop(0, ROWS, pack_body, 0)
        pltpu.sync_copy(hist2, out_hbm.at[wid])

    return deg_kernel, ROWS * 128


def _make_agg(N, D, E):
    ET = E // _NW
    NCH = ET // _CHA
    NSB = NCH // _SB
    RS0 = -(-N // (8 * _NS)) * 8
    RSL = N - (_NS - 1) * RS0

    @functools.partial(
        pl.kernel,
        mesh=_mesh(),
        out_type=jax.ShapeDtypeStruct((_NC, N, D), jnp.float32),
        scratch_types=[
            pltpu.VMEM_SHARED((N, D), jnp.float32),
            pltpu.VMEM((2, 2, _SB, _CHA), jnp.int32),
            pltpu.VMEM((_NB, _CHA, D), jnp.float32),
        ] + [pltpu.SemaphoreType.DMA] * (2 * _NB + 1),
    )
    def agg_kernel(g_hbm, ei_hbm, out_hbm, acc, eidx, rows, *sems):
        gsem = sems[:_NB]
        ssem = sems[_NB:2 * _NB]
        esem = sems[2 * _NB]
        c = lax.axis_index("c")
        s = lax.axis_index("s")
        wid = c * _NS + s

        @pl.when(s < _NS - 1)
        def _():
            r0 = pl.multiple_of(s * RS0, 8)
            pltpu.sync_copy(g_hbm.at[pl.ds(r0, RS0)], acc.at[pl.ds(r0, RS0)])

        @pl.when(s == _NS - 1)
        def _():
            r0 = (_NS - 1) * RS0
            pltpu.sync_copy(g_hbm.at[pl.ds(r0, RSL)], acc.at[pl.ds(r0, RSL)])

        plsc.subcore_barrier()

        pltpu.sync_copy(ei_hbm.at[:, wid, 0], eidx.at[0])
        for sb in range(NSB):
            pb = sb % 2
            exb = eidx.at[pb]
            if sb + 1 < NSB:
                pltpu.async_copy(ei_hbm.at[:, wid, sb + 1],
                                 eidx.at[1 - pb], esem)
            for p in range(2):
                if sb > 0:
                    pltpu.make_async_copy(
                        rows.at[p], acc.at[exb.at[1, p]], ssem[p]).wait()
                pltpu.async_copy(g_hbm.at[exb.at[0, p]], rows.at[p], gsem[p])

            def chunk_body(j, carry):
                for b in range(_NB):
                    i = j * _NB + b
                    b2 = (b + 2) % _NB

                    @pl.when(i + 2 < _SB)
                    def _():
                        def free_and_gather():
                            pltpu.make_async_copy(
                                rows.at[b2], acc.at[exb.at[1, i + 2 - _NB]],
                                ssem[b2]).wait()
                            pltpu.async_copy(
                                g_hbm.at[exb.at[0, i + 2]], rows.at[b2],
                                gsem[b2])
                        if sb == 0:
                            @pl.when(i + 2 >= _NB)
                            def _():
                                free_and_gather()

                            @pl.when(i + 2 < _NB)
                            def _():
                                pltpu.async_copy(
                                    g_hbm.at[exb.at[0, i + 2]], rows.at[b2],
                                    gsem[b2])
                        else:
                            free_and_gather()

                    pltpu.make_async_copy(
                        g_hbm.at[exb.at[0, i]], rows.at[b], gsem[b]).wait()
                    pltpu.async_copy(
                        rows.at[b], acc.at[exb.at[1, i]], ssem[b], add=True)
                return carry

            lax.fori_loop(0, _SB // _NB, chunk_body, 0)

            if sb + 1 < NSB:
                pltpu.make_async_copy(ei_hbm.at[:, wid, sb + 1],
                                      eidx.at[1 - pb], esem).wait()

        lastb = eidx.at[(NSB - 1) % 2]
        for b in range(_NB):
            pltpu.make_async_copy(
                rows.at[b], acc.at[lastb.at[1, _SB - _NB + b]],
                ssem[b]).wait()

        plsc.subcore_barrier()

        @pl.when(s < _NS - 1)
        def _():
            r0 = pl.multiple_of(s * RS0, 8)
            pltpu.sync_copy(acc.at[pl.ds(r0, RS0)],
                            out_hbm.at[c, pl.ds(r0, RS0)])

        @pl.when(s == _NS - 1)
        def _():
            r0 = (_NS - 1) * RS0
            pltpu.sync_copy(acc.at[pl.ds(r0, RSL)],
                            out_hbm.at[c, pl.ds(r0, RSL)])

    return agg_kernel


_PREC = jax.lax.Precision.HIGHEST


def _tc_pre(x, W, cnt, R=2000):
    N, D = x.shape
    H = W.shape[1]

    def body(x_ref, w_ref, c_ref, g_ref):
        dis = lax.rsqrt(c_ref[...] + 1.0)
        h = jnp.dot(x_ref[...], w_ref[...],
                    preferred_element_type=jnp.float32, precision=_PREC)
        g_ref[...] = h * dis

    return pl.pallas_call(
        body,
        grid=(N // R,),
        in_specs=[
            pl.BlockSpec((R, D), lambda i: (i, 0)),
            pl.BlockSpec((D, H), lambda i: (0, 0)),
            pl.BlockSpec((R, 1), lambda i: (i, 0)),
        ],
        out_specs=pl.BlockSpec((R, H), lambda i: (i, 0)),
        out_shape=jax.ShapeDtypeStruct((N, H), jnp.float32),
    )(x, W, cnt)


def _tc_mid(a, g, cnt, b, W, R=2000):
    _, N, H = a.shape
    H2 = W.shape[1]

    def body(a_ref, g_ref, c_ref, b_ref, w_ref, o_ref):
        dis = lax.rsqrt(c_ref[...] + 1.0)
        z = (a_ref[0] + a_ref[1] - g_ref[...]) * dis + b_ref[...]
        h = jnp.maximum(z, 0.0)
        o_ref[...] = jnp.dot(h, w_ref[...],
                             preferred_element_type=jnp.float32,
                             precision=_PREC) * dis

    return pl.pallas_call(
        body,
        grid=(N // R,),
        in_specs=[
            pl.BlockSpec((2, R, H), lambda i: (0, i, 0)),
            pl.BlockSpec((R, H), lambda i: (i, 0)),
            pl.BlockSpec((R, 1), lambda i: (i, 0)),
            pl.BlockSpec((1, H), lambda i: (0, 0)),
            pl.BlockSpec((H, H2), lambda i: (0, 0)),
        ],
        out_specs=pl.BlockSpec((R, H2), lambda i: (i, 0)),
        out_shape=jax.ShapeDtypeStruct((N, H2), jnp.float32),
    )(a, g, cnt, b, W)


def _tc_post(a, g, cnt, b, Wp1, bp1, Wp2, bp2, R=2000):
    _, N, H = a.shape
    O = Wp2.shape[1]

    def body(a_ref, g_ref, c_ref, b_ref, w1_ref, b1_ref, w2_ref, b2_ref,
             emb_ref, lsm_ref):
        dis = lax.rsqrt(c_ref[...] + 1.0)
        z = (a_ref[0] + a_ref[1] - g_ref[...]) * dis + b_ref[...]
        emb_ref[...] = z
        h = jnp.maximum(z, 0.0)
        t = jnp.dot(h, w1_ref[...], preferred_element_type=jnp.float32,
                    precision=_PREC) + b1_ref[...]
        o = jnp.dot(t, w2_ref[...], preferred_element_type=jnp.float32,
                    precision=_PREC) + b2_ref[...]
        m = jnp.max(o, axis=1, keepdims=True)
        lse = m + jnp.log(jnp.sum(jnp.exp(o - m), axis=1, keepdims=True))
        lsm_ref[...] = o - lse

    return pl.pallas_call(
        body,
        grid=(N // R,),
        in_specs=[
            pl.BlockSpec((2, R, H), lambda i: (0, i, 0)),
            pl.BlockSpec((R, H), lambda i: (i, 0)),
            pl.BlockSpec((R, 1), lambda i: (i, 0)),
            pl.BlockSpec((1, H), lambda i: (0, 0)),
            pl.BlockSpec((H, H), lambda i: (0, 0)),
            pl.BlockSpec((1, H), lambda i: (0, 0)),
            pl.BlockSpec((H, O), lambda i: (0, 0)),
            pl.BlockSpec((1, O), lambda i: (0, 0)),
        ],
        out_specs=[
            pl.BlockSpec((R, H), lambda i: (i, 0)),
            pl.BlockSpec((R, O), lambda i: (i, 0)),
        ],
        out_shape=[
            jax.ShapeDtypeStruct((N, H), jnp.float32),
            jax.ShapeDtypeStruct((N, O), jnp.float32),
        ],
    )(a, g, cnt, b, Wp1, bp1, Wp2, bp2)


def kernel(x, edge_index, batch, W1, b1, W2, b2, W3, b3, Wp1, bp1, Wp2, bp2):
    N, D = x.shape
    E = edge_index.shape[1]
    ET = E // _NW

    ei4 = edge_index.reshape(2, _NW, ET // _CH, _CH)
    ei5 = edge_index.reshape(2, _NW, ET // (_SB * _CHA), _SB, _CHA)

    deg_kernel, P = _make_deg(N, E)
    agg_kernel = _make_agg(N, D, E)

    cnt2 = deg_kernel(ei4)
    cnt = cnt2.sum(axis=0).reshape(-1)[:N].reshape(N, 1)

    g1 = _tc_pre(x, W1, cnt)
    a1 = agg_kernel(g1, ei5)
    g2 = _tc_mid(a1, g1, cnt, b1.reshape(1, -1), W2)
    a2 = agg_kernel(g2, ei5)
    g3 = _tc_mid(a2, g2, cnt, b2.reshape(1, -1), W3)
    a3 = agg_kernel(g3, ei5)
    emb, lsm = _tc_post(a3, g3, cnt, b3.reshape(1, -1),
                        Wp1, bp1.reshape(1, -1), Wp2, bp2.reshape(1, -1))
    return emb, lsm


# speedup vs baseline: 28.1223x; 1.0614x over previous
import functools

import jax
import jax.numpy as jnp
from jax import lax
from jax.experimental import pallas as pl
from jax.experimental.pallas import tpu as pltpu
from jax.experimental.pallas import tpu_sc as plsc

_NC = 2
_NS = 16
_NW = _NC * _NS
_CH = 80
_CHA = 40
_SB = 25
_NB = 5
_LA = 3


def _mesh():
    return plsc.VectorSubcoreMesh(core_axis_name="c", subcore_axis_name="s")


def _make_deg(N, E):
    ET = E // _NW
    NCH = ET // _CH
    ROWS = -(-N // (128 * _NS)) * _NS
    RT = ROWS // _NS

    @functools.partial(
        pl.kernel,
        mesh=_mesh(),
        out_type=jax.ShapeDtypeStruct((_NW, ROWS, 128), jnp.float32),
        compiler_params=pltpu.CompilerParams(needs_layout_passes=False),
        scratch_types=[
            pltpu.VMEM((ROWS * 128,), jnp.float32),
            pltpu.VMEM((ROWS, 128), jnp.float32),
            pltpu.VMEM((NCH, _CH), jnp.int32),
        ],
    )
    def deg_kernel(ei_hbm, out_hbm, hist, hist2, didx):
        c = lax.axis_index("c")
        s = lax.axis_index("s")
        wid = c * _NS + s

        zeros16 = jnp.zeros((16,), jnp.float32)
        lanes = lax.iota(jnp.int32, 16)

        def zero_body(r, carry):
            for k in range(8):
                hist[pl.ds(r * 128 + k * 16, 16)] = zeros16
            return carry

        lax.fori_loop(0, ROWS, zero_body, 0)

        pltpu.sync_copy(ei_hbm.at[1, wid], didx)

        ones16 = jnp.ones((16,), jnp.float32)

        def chunk_body(i, carry):
            for k in range(_CH // 16):
                idx = didx[i, pl.ds(k * 16, 16)]
                for j in range(16):
                    plsc.addupdate_scatter(
                        hist, [idx], ones16, mask=lanes == j)
            return carry

        lax.fori_loop(0, NCH, chunk_body, 0)

        def pack_body(r, carry):
            for k in range(8):
                hist2[r, pl.ds(k * 16, 16)] = hist[pl.ds(r * 128 + k * 16, 16)]
            return carry

        lax.fori_loop(0, ROWS, pack_body, 0)
        pltpu.sync_copy(hist2, out_hbm.at[wid])

    return deg_kernel, ROWS * 128


def _make_agg(N, D, E):
    ET = E // _NW
    NCH = ET // _CHA
    NSB = NCH // _SB
    RS0 = -(-N // (8 * _NS)) * 8
    RSL = N - (_NS - 1) * RS0

    @functools.partial(
        pl.kernel,
        mesh=_mesh(),
        out_type=jax.ShapeDtypeStruct((_NC, N, D), jnp.float32),
        scratch_types=[
            pltpu.VMEM_SHARED((N, D), jnp.float32),
            pltpu.VMEM((2, 2, _SB, _CHA), jnp.int32),
            pltpu.VMEM((_NB, _CHA, D), jnp.float32),
        ] + [pltpu.SemaphoreType.DMA] * (2 * _NB + 1),
    )
    def agg_kernel(g_hbm, ei_hbm, out_hbm, acc, eidx, rows, *sems):
        gsem = sems[:_NB]
        ssem = sems[_NB:2 * _NB]
        esem = sems[2 * _NB]
        c = lax.axis_index("c")
        s = lax.axis_index("s")
        wid = c * _NS + s

        @pl.when(s < _NS - 1)
        def _():
            r0 = pl.multiple_of(s * RS0, 8)
            pltpu.sync_copy(g_hbm.at[pl.ds(r0, RS0)], acc.at[pl.ds(r0, RS0)])

        @pl.when(s == _NS - 1)
        def _():
            r0 = (_NS - 1) * RS0
            pltpu.sync_copy(g_hbm.at[pl.ds(r0, RSL)], acc.at[pl.ds(r0, RSL)])

        plsc.subcore_barrier()

        pltpu.sync_copy(ei_hbm.at[:, wid, 0], eidx.at[0])
        for sb in range(NSB):
            pb = sb % 2
            exb = eidx.at[pb]
            if sb + 1 < NSB:
                pltpu.async_copy(ei_hbm.at[:, wid, sb + 1],
                                 eidx.at[1 - pb], esem)
            for p in range(_LA):
                if sb > 0:
                    pltpu.make_async_copy(
                        rows.at[p], acc.at[exb.at[1, p]], ssem[p]).wait()
                pltpu.async_copy(g_hbm.at[exb.at[0, p]], rows.at[p], gsem[p])

            def chunk_body(j, carry):
                for b in range(_NB):
                    i = j * _NB + b
                    b2 = (b + _LA) % _NB

                    @pl.when(i + _LA < _SB)
                    def _():
                        def free_and_gather():
                            pltpu.make_async_copy(
                                rows.at[b2], acc.at[exb.at[1, i + _LA - _NB]],
                                ssem[b2]).wait()
                            pltpu.async_copy(
                                g_hbm.at[exb.at[0, i + _LA]], rows.at[b2],
                                gsem[b2])
                        if sb == 0:
                            @pl.when(i + _LA >= _NB)
                            def _():
                                free_and_gather()

                            @pl.when(i + _LA < _NB)
                            def _():
                                pltpu.async_copy(
                                    g_hbm.at[exb.at[0, i + _LA]], rows.at[b2],
                                    gsem[b2])
                        else:
                            free_and_gather()

                    pltpu.make_async_copy(
                        g_hbm.at[exb.at[0, i]], rows.at[b], gsem[b]).wait()
                    pltpu.async_copy(
                        rows.at[b], acc.at[exb.at[1, i]], ssem[b], add=True)
                return carry

            lax.fori_loop(0, _SB // _NB, chunk_body, 0)

            if sb + 1 < NSB:
                pltpu.make_async_copy(ei_hbm.at[:, wid, sb + 1],
                                      eidx.at[1 - pb], esem).wait()

        lastb = eidx.at[(NSB - 1) % 2]
        for b in range(_NB):
            pltpu.make_async_copy(
                rows.at[b], acc.at[lastb.at[1, _SB - _NB + b]],
                ssem[b]).wait()

        plsc.subcore_barrier()

        @pl.when(s < _NS - 1)
        def _():
            r0 = pl.multiple_of(s * RS0, 8)
            pltpu.sync_copy(acc.at[pl.ds(r0, RS0)],
                            out_hbm.at[c, pl.ds(r0, RS0)])

        @pl.when(s == _NS - 1)
        def _():
            r0 = (_NS - 1) * RS0
            pltpu.sync_copy(acc.at[pl.ds(r0, RSL)],
                            out_hbm.at[c, pl.ds(r0, RSL)])

    return agg_kernel


_PREC = jax.lax.Precision.HIGHEST


def _tc_pre(x, W, cnt, R=2000):
    N, D = x.shape
    H = W.shape[1]

    def body(x_ref, w_ref, c_ref, g_ref):
        dis = lax.rsqrt(c_ref[...] + 1.0)
        h = jnp.dot(x_ref[...], w_ref[...],
                    preferred_element_type=jnp.float32, precision=_PREC)
        g_ref[...] = h * dis

    return pl.pallas_call(
        body,
        grid=(N // R,),
        in_specs=[
            pl.BlockSpec((R, D), lambda i: (i, 0)),
            pl.BlockSpec((D, H), lambda i: (0, 0)),
            pl.BlockSpec((R, 1), lambda i: (i, 0)),
        ],
        out_specs=pl.BlockSpec((R, H), lambda i: (i, 0)),
        out_shape=jax.ShapeDtypeStruct((N, H), jnp.float32),
    )(x, W, cnt)


def _tc_mid(a, g, cnt, b, W, R=2000):
    _, N, H = a.shape
    H2 = W.shape[1]

    def body(a_ref, g_ref, c_ref, b_ref, w_ref, o_ref):
        dis = lax.rsqrt(c_ref[...] + 1.0)
        z = (a_ref[0] + a_ref[1] - g_ref[...]) * dis + b_ref[...]
        h = jnp.maximum(z, 0.0)
        o_ref[...] = jnp.dot(h, w_ref[...],
                             preferred_element_type=jnp.float32,
                             precision=_PREC) * dis

    return pl.pallas_call(
        body,
        grid=(N // R,),
        in_specs=[
            pl.BlockSpec((2, R, H), lambda i: (0, i, 0)),
            pl.BlockSpec((R, H), lambda i: (i, 0)),
            pl.BlockSpec((R, 1), lambda i: (i, 0)),
            pl.BlockSpec((1, H), lambda i: (0, 0)),
            pl.BlockSpec((H, H2), lambda i: (0, 0)),
        ],
        out_specs=pl.BlockSpec((R, H2), lambda i: (i, 0)),
        out_shape=jax.ShapeDtypeStruct((N, H2), jnp.float32),
    )(a, g, cnt, b, W)


def _tc_post(a, g, cnt, b, Wp1, bp1, Wp2, bp2, R=2000):
    _, N, H = a.shape
    O = Wp2.shape[1]

    def body(a_ref, g_ref, c_ref, b_ref, w1_ref, b1_ref, w2_ref, b2_ref,
             emb_ref, lsm_ref):
        dis = lax.rsqrt(c_ref[...] + 1.0)
        z = (a_ref[0] + a_ref[1] - g_ref[...]) * dis + b_ref[...]
        emb_ref[...] = z
        h = jnp.maximum(z, 0.0)
        t = jnp.dot(h, w1_ref[...], preferred_element_type=jnp.float32,
                    precision=_PREC) + b1_ref[...]
        o = jnp.dot(t, w2_ref[...], preferred_element_type=jnp.float32,
                    precision=_PREC) + b2_ref[...]
        m = jnp.max(o, axis=1, keepdims=True)
        lse = m + jnp.log(jnp.sum(jnp.exp(o - m), axis=1, keepdims=True))
        lsm_ref[...] = o - lse

    return pl.pallas_call(
        body,
        grid=(N // R,),
        in_specs=[
            pl.BlockSpec((2, R, H), lambda i: (0, i, 0)),
            pl.BlockSpec((R, H), lambda i: (i, 0)),
            pl.BlockSpec((R, 1), lambda i: (i, 0)),
            pl.BlockSpec((1, H), lambda i: (0, 0)),
            pl.BlockSpec((H, H), lambda i: (0, 0)),
            pl.BlockSpec((1, H), lambda i: (0, 0)),
            pl.BlockSpec((H, O), lambda i: (0, 0)),
            pl.BlockSpec((1, O), lambda i: (0, 0)),
        ],
        out_specs=[
            pl.BlockSpec((R, H), lambda i: (i, 0)),
            pl.BlockSpec((R, O), lambda i: (i, 0)),
        ],
        out_shape=[
            jax.ShapeDtypeStruct((N, H), jnp.float32),
            jax.ShapeDtypeStruct((N, O), jnp.float32),
        ],
    )(a, g, cnt, b, Wp1, bp1, Wp2, bp2)


def kernel(x, edge_index, batch, W1, b1, W2, b2, W3, b3, Wp1, bp1, Wp2, bp2):
    N, D = x.shape
    E = edge_index.shape[1]
    ET = E // _NW

    ei4 = edge_index.reshape(2, _NW, ET // _CH, _CH)
    ei5 = edge_index.reshape(2, _NW, ET // (_SB * _CHA), _SB, _CHA)

    deg_kernel, P = _make_deg(N, E)
    agg_kernel = _make_agg(N, D, E)

    cnt2 = deg_kernel(ei4)
    cnt = cnt2.sum(axis=0).reshape(-1)[:N].reshape(N, 1)

    g1 = _tc_pre(x, W1, cnt)
    a1 = agg_kernel(g1, ei5)
    g2 = _tc_mid(a1, g1, cnt, b1.reshape(1, -1), W2)
    a2 = agg_kernel(g2, ei5)
    g3 = _tc_mid(a2, g2, cnt, b2.reshape(1, -1), W3)
    a3 = agg_kernel(g3, ei5)
    emb, lsm = _tc_post(a3, g3, cnt, b3.reshape(1, -1),
                        Wp1, bp1.reshape(1, -1), Wp2, bp2.reshape(1, -1))
    return emb, lsm


# speedup vs baseline: 28.4782x; 1.0127x over previous
import functools

import jax
import jax.numpy as jnp
from jax import lax
from jax.experimental import pallas as pl
from jax.experimental.pallas import tpu as pltpu
from jax.experimental.pallas import tpu_sc as plsc

_NC = 2
_NS = 16
_NW = _NC * _NS
_CH = 80
_CHA = 40
_SB = 25
_NB = 5
_LA = 4


def _mesh():
    return plsc.VectorSubcoreMesh(core_axis_name="c", subcore_axis_name="s")


def _make_deg(N, E):
    ET = E // _NW
    NCH = ET // _CH
    ROWS = -(-N // (128 * _NS)) * _NS
    RT = ROWS // _NS

    @functools.partial(
        pl.kernel,
        mesh=_mesh(),
        out_type=jax.ShapeDtypeStruct((_NW, ROWS, 128), jnp.float32),
        compiler_params=pltpu.CompilerParams(needs_layout_passes=False),
        scratch_types=[
            pltpu.VMEM((ROWS * 128,), jnp.float32),
            pltpu.VMEM((ROWS, 128), jnp.float32),
            pltpu.VMEM((NCH, _CH), jnp.int32),
        ],
    )
    def deg_kernel(ei_hbm, out_hbm, hist, hist2, didx):
        c = lax.axis_index("c")
        s = lax.axis_index("s")
        wid = c * _NS + s

        zeros16 = jnp.zeros((16,), jnp.float32)
        lanes = lax.iota(jnp.int32, 16)

        def zero_body(r, carry):
            for k in range(8):
                hist[pl.ds(r * 128 + k * 16, 16)] = zeros16
            return carry

        lax.fori_loop(0, ROWS, zero_body, 0)

        pltpu.sync_copy(ei_hbm.at[1, wid], didx)

        ones16 = jnp.ones((16,), jnp.float32)

        def chunk_body(i, carry):
            for k in range(_CH // 16):
                idx = didx[i, pl.ds(k * 16, 16)]
                for j in range(16):
                    plsc.addupdate_scatter(
                        hist, [idx], ones16, mask=lanes == j)
            return carry

        lax.fori_loop(0, NCH, chunk_body, 0)

        def pack_body(r, carry):
            for k in range(8):
                hist2[r, pl.ds(k * 16, 16)] = hist[pl.ds(r * 128 + k * 16, 16)]
            return carry

        lax.fori_loop(0, ROWS, pack_body, 0)
        pltpu.sync_copy(hist2, out_hbm.at[wid])

    return deg_kernel, ROWS * 128


def _make_agg(N, D, E):
    ET = E // _NW
    NCH = ET // _CHA
    NSB = NCH // _SB
    RS0 = -(-N // (8 * _NS)) * 8
    RSL = N - (_NS - 1) * RS0

    @functools.partial(
        pl.kernel,
        mesh=_mesh(),
        out_type=jax.ShapeDtypeStruct((_NC, N, D), jnp.float32),
        scratch_types=[
            pltpu.VMEM_SHARED((N, D), jnp.float32),
            pltpu.VMEM((2, 2, _SB, _CHA), jnp.int32),
            pltpu.VMEM((_NB, _CHA, D), jnp.float32),
        ] + [pltpu.SemaphoreType.DMA] * (2 * _NB + 1),
    )
    def agg_kernel(g_hbm, ei_hbm, out_hbm, acc, eidx, rows, *sems):
        gsem = sems[:_NB]
        ssem = sems[_NB:2 * _NB]
        esem = sems[2 * _NB]
        c = lax.axis_index("c")
        s = lax.axis_index("s")
        wid = c * _NS + s

        @pl.when(s < _NS - 1)
        def _():
            r0 = pl.multiple_of(s * RS0, 8)
            pltpu.sync_copy(g_hbm.at[pl.ds(r0, RS0)], acc.at[pl.ds(r0, RS0)])

        @pl.when(s == _NS - 1)
        def _():
            r0 = (_NS - 1) * RS0
            pltpu.sync_copy(g_hbm.at[pl.ds(r0, RSL)], acc.at[pl.ds(r0, RSL)])

        plsc.subcore_barrier()

        pltpu.sync_copy(ei_hbm.at[:, wid, 0], eidx.at[0])
        for sb in range(NSB):
            pb = sb % 2
            exb = eidx.at[pb]
            if sb + 1 < NSB:
                pltpu.async_copy(ei_hbm.at[:, wid, sb + 1],
                                 eidx.at[1 - pb], esem)
            for p in range(_LA):
                if sb > 0:
                    pltpu.make_async_copy(
                        rows.at[p], acc.at[exb.at[1, p]], ssem[p]).wait()
                pltpu.async_copy(g_hbm.at[exb.at[0, p]], rows.at[p], gsem[p])

            def chunk_body(j, carry):
                for b in range(_NB):
                    i = j * _NB + b
                    b2 = (b + _LA) % _NB

                    @pl.when(i + _LA < _SB)
                    def _():
                        def free_and_gather():
                            pltpu.make_async_copy(
                                rows.at[b2], acc.at[exb.at[1, i + _LA - _NB]],
                                ssem[b2]).wait()
                            pltpu.async_copy(
                                g_hbm.at[exb.at[0, i + _LA]], rows.at[b2],
                                gsem[b2])
                        if sb == 0:
                            @pl.when(i + _LA >= _NB)
                            def _():
                                free_and_gather()

                            @pl.when(i + _LA < _NB)
                            def _():
                                pltpu.async_copy(
                                    g_hbm.at[exb.at[0, i + _LA]], rows.at[b2],
                                    gsem[b2])
                        else:
                            free_and_gather()

                    pltpu.make_async_copy(
                        g_hbm.at[exb.at[0, i]], rows.at[b], gsem[b]).wait()
                    pltpu.async_copy(
                        rows.at[b], acc.at[exb.at[1, i]], ssem[b], add=True)
                return carry

            lax.fori_loop(0, _SB // _NB, chunk_body, 0)

            if sb + 1 < NSB:
                pltpu.make_async_copy(ei_hbm.at[:, wid, sb + 1],
                                      eidx.at[1 - pb], esem).wait()

        lastb = eidx.at[(NSB - 1) % 2]
        for b in range(_NB):
            pltpu.make_async_copy(
                rows.at[b], acc.at[lastb.at[1, _SB - _NB + b]],
                ssem[b]).wait()

        plsc.subcore_barrier()

        @pl.when(s < _NS - 1)
        def _():
            r0 = pl.multiple_of(s * RS0, 8)
            pltpu.sync_copy(acc.at[pl.ds(r0, RS0)],
                            out_hbm.at[c, pl.ds(r0, RS0)])

        @pl.when(s == _NS - 1)
        def _():
            r0 = (_NS - 1) * RS0
            pltpu.sync_copy(acc.at[pl.ds(r0, RSL)],
                            out_hbm.at[c, pl.ds(r0, RSL)])

    return agg_kernel


_PREC = jax.lax.Precision.HIGHEST


def _tc_pre(x, W, cnt, R=2000):
    N, D = x.shape
    H = W.shape[1]

    def body(x_ref, w_ref, c_ref, g_ref):
        dis = lax.rsqrt(c_ref[...] + 1.0)
        h = jnp.dot(x_ref[...], w_ref[...],
                    preferred_element_type=jnp.float32, precision=_PREC)
        g_ref[...] = h * dis

    return pl.pallas_call(
        body,
        grid=(N // R,),
        in_specs=[
            pl.BlockSpec((R, D), lambda i: (i, 0)),
            pl.BlockSpec((D, H), lambda i: (0, 0)),
            pl.BlockSpec((R, 1), lambda i: (i, 0)),
        ],
        out_specs=pl.BlockSpec((R, H), lambda i: (i, 0)),
        out_shape=jax.ShapeDtypeStruct((N, H), jnp.float32),
    )(x, W, cnt)


def _tc_mid(a, g, cnt, b, W, R=2000):
    _, N, H = a.shape
    H2 = W.shape[1]

    def body(a_ref, g_ref, c_ref, b_ref, w_ref, o_ref):
        dis = lax.rsqrt(c_ref[...] + 1.0)
        z = (a_ref[0] + a_ref[1] - g_ref[...]) * dis + b_ref[...]
        h = jnp.maximum(z, 0.0)
        o_ref[...] = jnp.dot(h, w_ref[...],
                             preferred_element_type=jnp.float32,
                             precision=_PREC) * dis

    return pl.pallas_call(
        body,
        grid=(N // R,),
        in_specs=[
            pl.BlockSpec((2, R, H), lambda i: (0, i, 0)),
            pl.BlockSpec((R, H), lambda i: (i, 0)),
            pl.BlockSpec((R, 1), lambda i: (i, 0)),
            pl.BlockSpec((1, H), lambda i: (0, 0)),
            pl.BlockSpec((H, H2), lambda i: (0, 0)),
        ],
        out_specs=pl.BlockSpec((R, H2), lambda i: (i, 0)),
        out_shape=jax.ShapeDtypeStruct((N, H2), jnp.float32),
    )(a, g, cnt, b, W)


def _tc_post(a, g, cnt, b, Wp1, bp1, Wp2, bp2, R=2000):
    _, N, H = a.shape
    O = Wp2.shape[1]

    def body(a_ref, g_ref, c_ref, b_ref, w1_ref, b1_ref, w2_ref, b2_ref,
             emb_ref, lsm_ref):
        dis = lax.rsqrt(c_ref[...] + 1.0)
        z = (a_ref[0] + a_ref[1] - g_ref[...]) * dis + b_ref[...]
        emb_ref[...] = z
        h = jnp.maximum(z, 0.0)
        t = jnp.dot(h, w1_ref[...], preferred_element_type=jnp.float32,
                    precision=_PREC) + b1_ref[...]
        o = jnp.dot(t, w2_ref[...], preferred_element_type=jnp.float32,
                    precision=_PREC) + b2_ref[...]
        m = jnp.max(o, axis=1, keepdims=True)
        lse = m + jnp.log(jnp.sum(jnp.exp(o - m), axis=1, keepdims=True))
        lsm_ref[...] = o - lse

    return pl.pallas_call(
        body,
        grid=(N // R,),
        in_specs=[
            pl.BlockSpec((2, R, H), lambda i: (0, i, 0)),
            pl.BlockSpec((R, H), lambda i: (i, 0)),
            pl.BlockSpec((R, 1), lambda i: (i, 0)),
            pl.BlockSpec((1, H), lambda i: (0, 0)),
            pl.BlockSpec((H, H), lambda i: (0, 0)),
            pl.BlockSpec((1, H), lambda i: (0, 0)),
            pl.BlockSpec((H, O), lambda i: (0, 0)),
            pl.BlockSpec((1, O), lambda i: (0, 0)),
        ],
        out_specs=[
            pl.BlockSpec((R, H), lambda i: (i, 0)),
            pl.BlockSpec((R, O), lambda i: (i, 0)),
        ],
        out_shape=[
            jax.ShapeDtypeStruct((N, H), jnp.float32),
            jax.ShapeDtypeStruct((N, O), jnp.float32),
        ],
    )(a, g, cnt, b, Wp1, bp1, Wp2, bp2)


def kernel(x, edge_index, batch, W1, b1, W2, b2, W3, b3, Wp1, bp1, Wp2, bp2):
    N, D = x.shape
    E = edge_index.shape[1]
    ET = E // _NW

    ei4 = edge_index.reshape(2, _NW, ET // _CH, _CH)
    ei5 = edge_index.reshape(2, _NW, ET // (_SB * _CHA), _SB, _CHA)

    deg_kernel, P = _make_deg(N, E)
    agg_kernel = _make_agg(N, D, E)

    cnt2 = deg_kernel(ei4)
    cnt = cnt2.sum(axis=0).reshape(-1)[:N].reshape(N, 1)

    g1 = _tc_pre(x, W1, cnt)
    a1 = agg_kernel(g1, ei5)
    g2 = _tc_mid(a1, g1, cnt, b1.reshape(1, -1), W2)
    a2 = agg_kernel(g2, ei5)
    g3 = _tc_mid(a2, g2, cnt, b2.reshape(1, -1), W3)
    a3 = agg_kernel(g3, ei5)
    emb, lsm = _tc_post(a3, g3, cnt, b3.reshape(1, -1),
                        Wp1, bp1.reshape(1, -1), Wp2, bp2.reshape(1, -1))
    return emb, lsm
